# baseline (speedup 1.0000x reference)
import functools

import jax
import jax.numpy as jnp
from jax import lax
from jax.experimental import pallas as pl
from jax.experimental.pallas import tpu as pltpu
from jax.experimental.pallas import tpu_sc as plsc

_D = 64
_NC = 2
_NS = 16
_NW = _NC * _NS


@functools.lru_cache(maxsize=None)
def _build(B: int, C: int):
    b_per_w = B // _NW
    n_chunks = b_per_w // C
    mesh = plsc.VectorSubcoreMesh(core_axis_name="c", subcore_axis_name="s")

    @functools.partial(
        pl.kernel,
        mesh=mesh,
        out_type=jax.ShapeDtypeStruct((B, _D), jnp.float32),
        scratch_types=[
            pltpu.VMEM((C,), jnp.int32),
            pltpu.VMEM((C, _D), jnp.float32),
            pltpu.SemaphoreType.DMA,
        ],
        compiler_params=pltpu.CompilerParams(use_tc_tiling_on_sc=False),
    )
    def gather_kernel(table_hbm, idx_hbm, out_hbm, idx_v, rows_v, sem):
        wid = lax.axis_index("s") * _NC + lax.axis_index("c")
        base = wid * b_per_w

        def body(i, carry):
            off = base + i * C
            pltpu.sync_copy(idx_hbm.at[pl.ds(off, C)], idx_v)
            pltpu.async_copy(table_hbm.at[idx_v], rows_v, sem).wait()
            pltpu.sync_copy(rows_v, out_hbm.at[pl.ds(off, C)])
            return carry

        lax.fori_loop(0, n_chunks, body, 0)

    return gather_kernel


def kernel(seq, table):
    s0, s1 = seq.shape
    b = s0 * s1
    flat = seq.reshape(b).astype(jnp.int32)
    out = _build(b, 1024)(table, flat)
    return out.reshape(s0, s1, _D)


# speedup vs baseline: 1.0209x; 1.0209x over previous
import functools

import jax
import jax.numpy as jnp
from jax import lax
from jax.experimental import pallas as pl
from jax.experimental.pallas import tpu as pltpu
from jax.experimental.pallas import tpu_sc as plsc

_D = 64
_NC = 2
_NS = 16
_NW = _NC * _NS


@functools.lru_cache(maxsize=None)
def _build(B: int, C: int):
    b_per_w = B // _NW
    n = b_per_w // C
    assert n % 2 == 0 and n >= 4
    mesh = plsc.VectorSubcoreMesh(core_axis_name="c", subcore_axis_name="s")

    @functools.partial(
        pl.kernel,
        mesh=mesh,
        out_type=jax.ShapeDtypeStruct((B, _D), jnp.float32),
        scratch_types=[
            pltpu.VMEM((2, C), jnp.int32),
            pltpu.VMEM((2, C, _D), jnp.float32),
            pltpu.SemaphoreType.DMA((2,)),
            pltpu.SemaphoreType.DMA((2,)),
        ],
        compiler_params=pltpu.CompilerParams(use_tc_tiling_on_sc=False),
    )
    def gather_kernel(table_hbm, idx_hbm, out_hbm, idx_v, rows_v, gsem, wsem):
        wid = lax.axis_index("s") * _NC + lax.axis_index("c")
        base = wid * b_per_w

        for b in range(2):
            pltpu.sync_copy(idx_hbm.at[pl.ds(base + b * C, C)], idx_v.at[b])
            pltpu.async_copy(table_hbm.at[idx_v.at[b]], rows_v.at[b],
                             gsem.at[b])

        @pl.loop(0, n - 2, step=2)
        def _pair(g):
            for b in range(2):
                i = g + b
                off = base + i * C
                pltpu.make_async_copy(table_hbm.at[idx_v.at[b]],
                                      rows_v.at[b], gsem.at[b]).wait()
                pltpu.async_copy(rows_v.at[b], out_hbm.at[pl.ds(off, C)],
                                 wsem.at[b])
                pltpu.sync_copy(idx_hbm.at[pl.ds(off + 2 * C, C)],
                                idx_v.at[b])
                pltpu.make_async_copy(rows_v.at[b],
                                      out_hbm.at[pl.ds(off, C)],
                                      wsem.at[b]).wait()
                pltpu.async_copy(table_hbm.at[idx_v.at[b]], rows_v.at[b],
                                 gsem.at[b])

        for b in range(2):
            i = n - 2 + b
            off = base + i * C
            pltpu.make_async_copy(table_hbm.at[idx_v.at[b]], rows_v.at[b],
                                  gsem.at[b]).wait()
            pltpu.sync_copy(rows_v.at[b], out_hbm.at[pl.ds(off, C)])

    return gather_kernel


def kernel(seq, table):
    s0, s1 = seq.shape
    b = s0 * s1
    flat = seq.reshape(b).astype(jnp.int32)
    out = _build(b, 512)(table, flat)
    return out.reshape(s0, s1, _D)
